# baseline (device time: 11983 ns/iter reference)
import jax
import jax.numpy as jnp
from jax import lax
from jax.experimental import pallas as pl
from jax.experimental.pallas import tpu as pltpu

N_CHUNK = 8


def kernel(x):
    m, n = x.shape
    ch = m // N_CHUNK

    def body(x_hbm, out_hbm, buf, acc, send_buf, recv_buf,
             copy_sems, out_sem, send_sem, recv_sem):
        my_x = lax.axis_index("x")
        my_y = lax.axis_index("y")
        nbr = (1 - my_x, my_y)

        barrier_sem = pltpu.get_barrier_semaphore()
        pl.semaphore_signal(
            barrier_sem, inc=1,
            device_id=nbr, device_id_type=pl.DeviceIdType.MESH,
        )

        def chunk_copy(k, slot):
            return pltpu.make_async_copy(
                x_hbm.at[pl.ds(k * ch, ch), :],
                buf.at[slot],
                copy_sems.at[slot],
            )

        chunk_copy(0, 0).start()
        chunk_copy(1, 1).start()
        for k in range(N_CHUNK):
            slot = k % 2
            chunk_copy(k, slot).wait()
            t = jnp.sum(buf[slot], axis=0, keepdims=True)
            if k == 0:
                acc[:, :] = t
            else:
                acc[:, :] += t
            if k + 2 < N_CHUNK:
                chunk_copy(k + 2, slot).start()

        send_buf[:, :] = acc[:, :]
        pl.semaphore_wait(barrier_sem, 1)

        rdma = pltpu.make_async_remote_copy(
            src_ref=send_buf,
            dst_ref=recv_buf,
            send_sem=send_sem,
            recv_sem=recv_sem,
            device_id=nbr,
            device_id_type=pl.DeviceIdType.MESH,
        )
        rdma.start()
        rdma.wait()

        out_hbm[:, :] = send_buf[:, :] + recv_buf[:, :]

    return pl.pallas_call(
        body,
        out_shape=jax.ShapeDtypeStruct((1, n), jnp.float32),
        in_specs=[pl.BlockSpec(memory_space=pltpu.MemorySpace.HBM)],
        out_specs=pl.BlockSpec(memory_space=pltpu.MemorySpace.VMEM),
        scratch_shapes=[
            pltpu.VMEM((2, ch, n), jnp.float32),
            pltpu.VMEM((1, n), jnp.float32),
            pltpu.VMEM((1, n), jnp.float32),
            pltpu.VMEM((1, n), jnp.float32),
            pltpu.SemaphoreType.DMA((2,)),
            pltpu.SemaphoreType.DMA,
            pltpu.SemaphoreType.DMA,
            pltpu.SemaphoreType.DMA,
        ],
        compiler_params=pltpu.CompilerParams(collective_id=0),
    )(x)


# device time: 11886 ns/iter; 1.0082x vs baseline; 1.0082x over previous
import jax
import jax.numpy as jnp
from jax import lax
from jax.experimental import pallas as pl
from jax.experimental.pallas import tpu as pltpu

N_CHUNK = 8


def kernel(x):
    m, n = x.shape
    ch = m // N_CHUNK

    def body(x_hbm, out_hbm, buf, acc, send_buf, recv_buf,
             copy_sems, out_sem, send_sem, recv_sem):
        my_x = lax.axis_index("x")
        my_y = lax.axis_index("y")
        nbr = (1 - my_x, my_y)

        barrier_sem = pltpu.get_barrier_semaphore()
        pl.semaphore_signal(
            barrier_sem, inc=1,
            device_id=nbr, device_id_type=pl.DeviceIdType.MESH,
        )

        def chunk_copy(k, slot):
            return pltpu.make_async_copy(
                x_hbm.at[pl.ds(k * ch, ch), :],
                buf.at[slot],
                copy_sems.at[slot],
            )

        chunk_copy(0, 0).start()
        chunk_copy(1, 1).start()
        for k in range(N_CHUNK):
            slot = k % 2
            chunk_copy(k, slot).wait()
            t = jnp.sum(buf[slot], axis=0, keepdims=True)
            if k == 0:
                acc[:, :] = t
            else:
                acc[:, :] += t
            if k + 2 < N_CHUNK:
                chunk_copy(k + 2, slot).start()

        send_buf[:, :] = acc[:, :]
        pl.semaphore_wait(barrier_sem, 1)

        rdma = pltpu.make_async_remote_copy(
            src_ref=send_buf,
            dst_ref=recv_buf,
            send_sem=send_sem,
            recv_sem=recv_sem,
            device_id=nbr,
            device_id_type=pl.DeviceIdType.MESH,
        )
        rdma.start()
        rdma.wait()

        out_hbm[:, :] = send_buf[:, :] + recv_buf[:, :]

    return pl.pallas_call(
        body,
        out_shape=jax.ShapeDtypeStruct((1, n), jnp.float32),
        in_specs=[pl.BlockSpec(memory_space=pltpu.MemorySpace.HBM)],
        out_specs=pl.BlockSpec(memory_space=pltpu.MemorySpace.VMEM),
        scratch_shapes=[
            pltpu.VMEM((2, ch, n), jnp.float32),
            pltpu.VMEM((1, n), jnp.float32),
            pltpu.VMEM((1, n), jnp.float32),
            pltpu.VMEM((1, n), jnp.float32),
            pltpu.SemaphoreType.DMA((2,)),
            pltpu.SemaphoreType.DMA,
            pltpu.SemaphoreType.DMA,
            pltpu.SemaphoreType.DMA,
        ],
        compiler_params=pltpu.CompilerParams(collective_id=0),
    )(pltpu.with_memory_space_constraint(x, pltpu.MemorySpace.HBM))


# device time: 10142 ns/iter; 1.1815x vs baseline; 1.1720x over previous
import jax
import jax.numpy as jnp
from jax import lax
from jax.experimental import pallas as pl
from jax.experimental.pallas import tpu as pltpu

N_CHUNK = 8
DEPTH = 4


def kernel(x):
    m, n = x.shape
    ch = m // N_CHUNK

    def body(x_hbm, out_hbm, buf, acc, send_buf, recv_buf,
             copy_sems, out_sem, send_sem, recv_sem):
        my_x = lax.axis_index("x")
        my_y = lax.axis_index("y")
        nbr = (1 - my_x, my_y)

        barrier_sem = pltpu.get_barrier_semaphore()
        pl.semaphore_signal(
            barrier_sem, inc=1,
            device_id=nbr, device_id_type=pl.DeviceIdType.MESH,
        )

        def chunk_copy(k, slot):
            return pltpu.make_async_copy(
                x_hbm.at[pl.ds(k * ch, ch), :],
                buf.at[slot],
                copy_sems.at[slot],
            )

        for d in range(DEPTH):
            chunk_copy(d, d).start()
        for k in range(N_CHUNK):
            slot = k % DEPTH
            chunk_copy(k, slot).wait()
            t = jnp.sum(buf[slot], axis=0, keepdims=True)
            if k == 0:
                acc[:, :] = t
            else:
                acc[:, :] += t
            if k + DEPTH < N_CHUNK:
                chunk_copy(k + DEPTH, slot).start()

        send_buf[:, :] = acc[:, :]
        pl.semaphore_wait(barrier_sem, 1)

        rdma = pltpu.make_async_remote_copy(
            src_ref=send_buf,
            dst_ref=recv_buf,
            send_sem=send_sem,
            recv_sem=recv_sem,
            device_id=nbr,
            device_id_type=pl.DeviceIdType.MESH,
        )
        rdma.start()
        rdma.wait()

        out_hbm[:, :] = send_buf[:, :] + recv_buf[:, :]

    return pl.pallas_call(
        body,
        out_shape=jax.ShapeDtypeStruct((1, n), jnp.float32),
        in_specs=[pl.BlockSpec(memory_space=pltpu.MemorySpace.HBM)],
        out_specs=pl.BlockSpec(memory_space=pltpu.MemorySpace.VMEM),
        scratch_shapes=[
            pltpu.VMEM((DEPTH, ch, n), jnp.float32),
            pltpu.VMEM((1, n), jnp.float32),
            pltpu.VMEM((1, n), jnp.float32),
            pltpu.VMEM((1, n), jnp.float32),
            pltpu.SemaphoreType.DMA((DEPTH,)),
            pltpu.SemaphoreType.DMA,
            pltpu.SemaphoreType.DMA,
            pltpu.SemaphoreType.DMA,
        ],
        compiler_params=pltpu.CompilerParams(collective_id=0),
    )(pltpu.with_memory_space_constraint(x, pltpu.MemorySpace.HBM))


# device time: 10131 ns/iter; 1.1828x vs baseline; 1.0011x over previous
import jax
import jax.numpy as jnp
from jax import lax
from jax.experimental import pallas as pl
from jax.experimental.pallas import tpu as pltpu

N_CHUNK = 8
DEPTH = 4


def kernel(x):
    m, n = x.shape
    ch = m // N_CHUNK

    def body(x_hbm, out_hbm, buf, acc, send_buf, recv_buf,
             copy_sems, out_sem, send_sem, recv_sem):
        my_x = lax.axis_index("x")
        my_y = lax.axis_index("y")
        nbr = (1 - my_x, my_y)

        barrier_sem = pltpu.get_barrier_semaphore()
        pl.semaphore_signal(
            barrier_sem, inc=1,
            device_id=nbr, device_id_type=pl.DeviceIdType.MESH,
        )

        def chunk_copy(k, slot):
            return pltpu.make_async_copy(
                x_hbm.at[pl.ds(k * ch, ch), :],
                buf.at[slot],
                copy_sems.at[slot],
            )

        for d in range(DEPTH):
            chunk_copy(d, d).start()
        for k in range(N_CHUNK):
            slot = k % DEPTH
            chunk_copy(k, slot).wait()
            t = jnp.sum(buf[slot], axis=0, keepdims=True)
            if k == 0:
                acc[:, :] = t
            else:
                acc[:, :] += t
            if k + DEPTH < N_CHUNK:
                chunk_copy(k + DEPTH, slot).start()

        send_buf[:, :] = acc[:, :]
        pl.semaphore_wait(barrier_sem, 1)

        rdma = pltpu.make_async_remote_copy(
            src_ref=send_buf,
            dst_ref=recv_buf,
            send_sem=send_sem,
            recv_sem=recv_sem,
            device_id=nbr,
            device_id_type=pl.DeviceIdType.MESH,
        )
        rdma.start()
        rdma.wait()

        acc[:, :] = send_buf[:, :] + recv_buf[:, :]
        out_copy = pltpu.make_async_copy(acc, out_hbm, out_sem)
        out_copy.start()
        out_copy.wait()

    return pl.pallas_call(
        body,
        out_shape=jax.ShapeDtypeStruct((1, n), jnp.float32),
        in_specs=[pl.BlockSpec(memory_space=pltpu.MemorySpace.HBM)],
        out_specs=pl.BlockSpec(memory_space=pltpu.MemorySpace.HBM),
        scratch_shapes=[
            pltpu.VMEM((DEPTH, ch, n), jnp.float32),
            pltpu.VMEM((1, n), jnp.float32),
            pltpu.VMEM((1, n), jnp.float32),
            pltpu.VMEM((1, n), jnp.float32),
            pltpu.SemaphoreType.DMA((DEPTH,)),
            pltpu.SemaphoreType.DMA,
            pltpu.SemaphoreType.DMA,
            pltpu.SemaphoreType.DMA,
        ],
        compiler_params=pltpu.CompilerParams(collective_id=0),
    )(pltpu.with_memory_space_constraint(x, pltpu.MemorySpace.HBM))
